# baseline (speedup 1.0000x reference)
import jax
import jax.numpy as jnp
from jax.experimental import pallas as pl
from jax.experimental.pallas import tpu as pltpu

_DNT = (((1,), (1,)), ((), ()))


def _mushin_body(inp_hbm, incT_hbm, wattr_hbm, wenc_ref, benc_ref, battr_ref,
                 wconv_ref, att_ref, bconv_ref, wout_ref, bout_ref, out_ref,
                 inp_v, incT_v, wattr_v, sem_inp, sem_inc, sem_wattr):
    f32 = jnp.float32
    heads, two_c = att_ref.shape
    c = two_c // 2

    cp_inp = pltpu.make_async_copy(inp_hbm, inp_v, sem_inp)
    cp_inc = pltpu.make_async_copy(incT_hbm, incT_v, sem_inc)
    cp_wattr = pltpu.make_async_copy(wattr_hbm, wattr_v, sem_wattr)
    cp_inp.start()
    cp_inc.start()
    cp_wattr.start()

    cp_inp.wait()
    x = jnp.dot(inp_v[...], wenc_ref[...], preferred_element_type=f32)
    x = jnp.maximum(x + benc_ref[...], 0.0)
    xls = [jnp.dot(x, wconv_ref[:, h * c:(h + 1) * c],
                   preferred_element_type=f32) for h in range(heads)]

    cp_inc.wait()
    incT = incT_v[...]
    maskT = incT > 0.0
    deg_n = jnp.sum(incT, axis=0, keepdims=True)
    inv_dn = jnp.where(deg_n > 0.0, 1.0 / deg_n, 0.0)
    deg_e = jnp.sum(incT, axis=1, keepdims=True)
    inv_de = jnp.where(deg_e > 0.0, 1.0 / deg_e, 0.0)

    cp_wattr.wait()
    he = jnp.dot(incT, wattr_v[...], preferred_element_type=f32)
    he = he + battr_ref[...]

    res = None
    for h in range(heads):
        ai = att_ref[h:h + 1, :c]
        aj = att_ref[h:h + 1, c:]
        bc = bconv_ref[:, h * c:(h + 1) * c]
        wo = wout_ref[h * c:(h + 1) * c, :]

        xl = xls[h]
        ea = jnp.dot(he, wconv_ref[:, h * c:(h + 1) * c],
                     preferred_element_type=f32)
        a_i = jax.lax.dot_general(ai, xl, _DNT,
                                  preferred_element_type=f32)
        a_e = jax.lax.dot_general(ea, aj, _DNT,
                                  preferred_element_type=f32)
        logit = a_i + a_e
        logit = jnp.where(logit >= 0.0, logit, 0.2 * logit)
        lmask = jnp.where(maskT, logit, -1e30)
        amax = jnp.max(lmask, axis=0, keepdims=True)
        amax = jnp.where(amax > -1e29, amax, 0.0)
        ex = jnp.where(maskT, jnp.exp(logit - amax), 0.0)
        den = jnp.sum(ex, axis=0, keepdims=True)
        rden = 1.0 / (den + 1e-16)
        alphaT = ex * rden
        alphaT_dn = ex * (rden * inv_dn)

        out_e = inv_de * jnp.dot(alphaT, xl,
                                 preferred_element_type=f32)
        g = jax.lax.dot_general(incT, alphaT_dn, _DNT,
                                preferred_element_type=f32)
        hf = jnp.dot(g, out_e, preferred_element_type=f32)
        hf = hf + deg_e * bc
        part = jnp.dot(hf, wo, preferred_element_type=f32)
        res = part if res is None else res + part

    out_ref[...] = res + bout_ref[...]


def kernel(input_features, incidence_matrix, W_enc, b_enc, W_attr, b_attr,
           W_conv, att, b_conv, W_out, b_out):
    n, in_feat = input_features.shape
    m = incidence_matrix.shape[1]
    emb = W_enc.shape[1]
    heads = att.shape[1]

    any_spec = pl.BlockSpec(memory_space=pltpu.MemorySpace.HBM)
    vmem_spec = pl.BlockSpec(memory_space=pltpu.MemorySpace.VMEM)
    return pl.pallas_call(
        _mushin_body,
        in_specs=[any_spec, any_spec, any_spec] + [vmem_spec] * 8,
        out_specs=vmem_spec,
        out_shape=jax.ShapeDtypeStruct((m, b_out.shape[0]), jnp.float32),
        scratch_shapes=[
            pltpu.VMEM((n, in_feat), jnp.float32),
            pltpu.VMEM((m, n), jnp.float32),
            pltpu.VMEM((n, emb), jnp.float32),
            pltpu.SemaphoreType.DMA,
            pltpu.SemaphoreType.DMA,
            pltpu.SemaphoreType.DMA,
        ],
    )(input_features, incidence_matrix.T, W_attr, W_enc,
      b_enc.reshape(1, emb), b_attr.reshape(1, emb), W_conv,
      att.reshape(heads, -1), b_conv.reshape(1, -1), W_out,
      b_out.reshape(1, -1))
